# Initial kernel scaffold; baseline (speedup 1.0000x reference)
import jax
import jax.numpy as jnp
from jax.experimental import pallas as pl


def kernel(x, a_rows, a_cols, a_vals, W):
    raise NotImplementedError("write your pallas kernel here")


# speedup vs baseline: 3.3843x; 3.3843x over previous
import functools

import jax
import jax.numpy as jnp
from jax import lax
from jax.experimental import pallas as pl
from jax.experimental.pallas import tpu as pltpu
from jax.experimental.pallas import tpu_sc as plsc

N = 10000
N_PAD = 10240
F = 128
LANES = 16
NC = 2
NS = 16
NW = NC * NS
CH = 128
ROWS_PER_SUB = N_PAD // NS


def _sc_propagate(xf, cols2d, rows2d, vals2d, zeros, chunks_per_w):
    mesh = plsc.VectorSubcoreMesh(core_axis_name="c", subcore_axis_name="s")

    @functools.partial(
        pl.kernel,
        out_type=jax.ShapeDtypeStruct((NC, N_PAD, F), jnp.float32),
        mesh=mesh,
        scratch_types=[
            pltpu.VMEM_SHARED((N_PAD, F), jnp.float32),
            pltpu.VMEM((chunks_per_w, CH), jnp.int32),
            pltpu.VMEM((chunks_per_w, CH), jnp.int32),
            pltpu.VMEM((chunks_per_w * CH + LANES,), jnp.float32),
            pltpu.VMEM((CH, F), jnp.float32),
            pltpu.SemaphoreType.DMA,
        ],
    )
    def k(xf_hbm, cols_hbm, rows_hbm, vals_hbm, zeros_hbm, out_hbm,
          acc, cols_v, rows_v, vals_v, buf, sem):
        cid = lax.axis_index("c")
        sid = lax.axis_index("s")
        wid = cid * NS + sid

        pltpu.sync_copy(zeros_hbm, acc.at[pl.ds(sid * ROWS_PER_SUB, ROWS_PER_SUB)])

        base = wid * chunks_per_w
        pltpu.sync_copy(cols_hbm.at[pl.ds(base, chunks_per_w)], cols_v)
        pltpu.sync_copy(rows_hbm.at[pl.ds(base, chunks_per_w)], rows_v)
        pltpu.sync_copy(vals_hbm.at[pl.ds(base * CH, chunks_per_w * CH)],
                        vals_v.at[pl.ds(0, chunks_per_w * CH)])

        plsc.subcore_barrier()

        def chunk_body(c, _):
            pltpu.async_copy(xf_hbm.at[cols_v.at[c]], buf, sem).wait()

            def edge_body(e, _):
                vb = vals_v[pl.ds(c * CH + e, LANES)][0]
                for j in range(F // LANES):
                    sl = pl.ds(j * LANES, LANES)
                    buf[e, sl] = buf[e, sl] * vb
                return 0

            lax.fori_loop(0, CH, edge_body, 0)

            pltpu.sync_copy(buf, acc.at[rows_v.at[c]], add=True)
            return 0

        lax.fori_loop(0, chunks_per_w, chunk_body, 0)

        plsc.subcore_barrier()

        pltpu.sync_copy(
            acc.at[pl.ds(sid * ROWS_PER_SUB, ROWS_PER_SUB)],
            out_hbm.at[cid, pl.ds(sid * ROWS_PER_SUB, ROWS_PER_SUB)],
        )

    return k(xf, cols2d, rows2d, vals2d, zeros)


def _combine_matmul_body(p_ref, w_ref, o_ref):
    p0 = p_ref[0]
    p1 = p_ref[1]
    o_ref[...] = (
        jnp.dot(p0, w_ref[...], preferred_element_type=jnp.float32,
                precision=lax.Precision.HIGHEST)
        + jnp.dot(p1, w_ref[...], preferred_element_type=jnp.float32,
                  precision=lax.Precision.HIGHEST)
    )


def _combine_matmul(partials, W):
    blk = 1000
    grid = (N // blk,)
    return pl.pallas_call(
        _combine_matmul_body,
        grid=grid,
        in_specs=[
            pl.BlockSpec((NC, blk, F), lambda i: (0, i, 0)),
            pl.BlockSpec((F, F), lambda i: (0, 0)),
        ],
        out_specs=pl.BlockSpec((blk, F), lambda i: (i, 0)),
        out_shape=jax.ShapeDtypeStruct((N, F), jnp.float32),
    )(partials, W)


def kernel(x, a_rows, a_cols, a_vals, W):
    batch, n, f_in = x.shape
    xf = x.reshape(n, f_in)
    nnz = a_rows.shape[0]

    span = NW * CH
    chunks_per_w = -(-nnz // span)
    chunks_per_w = -(-chunks_per_w // 8) * 8
    nnz_pad = span * chunks_per_w
    pad = nnz_pad - nnz
    rows_p = jnp.pad(a_rows, (0, pad))
    cols_p = jnp.pad(a_cols, (0, pad))
    vals_p = jnp.pad(a_vals, (0, pad))

    rows2d = rows_p.reshape(NW * chunks_per_w, CH)
    cols2d = cols_p.reshape(NW * chunks_per_w, CH)
    zeros = jnp.zeros((ROWS_PER_SUB, F), jnp.float32)

    partials = _sc_propagate(xf, cols2d, rows2d, vals_p, zeros, chunks_per_w)
    out = _combine_matmul(partials, W)
    return out.reshape(batch, n, W.shape[1])


# speedup vs baseline: 3.7052x; 1.0948x over previous
import functools

import jax
import jax.numpy as jnp
from jax import lax
from jax.experimental import pallas as pl
from jax.experimental.pallas import tpu as pltpu
from jax.experimental.pallas import tpu_sc as plsc

N = 10000
N_PAD = 10240
F = 128
LANES = 16
NC = 2
NS = 16
NW = NC * NS
CH = 128
ROWS_PER_SUB = N_PAD // NS


def _sc_propagate(xf, cols2d, rows2d, vals2d, zeros, chunks_per_w):
    mesh = plsc.VectorSubcoreMesh(core_axis_name="c", subcore_axis_name="s")

    @functools.partial(
        pl.kernel,
        out_type=jax.ShapeDtypeStruct((NC, N_PAD, F), jnp.float32),
        mesh=mesh,
        scratch_types=[
            pltpu.VMEM_SHARED((N_PAD, F), jnp.float32),
            pltpu.VMEM((chunks_per_w, CH), jnp.int32),
            pltpu.VMEM((chunks_per_w, CH), jnp.int32),
            pltpu.VMEM((chunks_per_w * CH,), jnp.float32),
            pltpu.VMEM((CH, F), jnp.float32),
            pltpu.SemaphoreType.DMA,
        ],
    )
    def k(xf_hbm, cols_hbm, rows_hbm, vals_hbm, zeros_hbm, out_hbm,
          acc, cols_v, rows_v, vals_v, buf, sem):
        cid = lax.axis_index("c")
        sid = lax.axis_index("s")
        wid = cid * NS + sid

        pltpu.sync_copy(zeros_hbm, acc.at[pl.ds(sid * ROWS_PER_SUB, ROWS_PER_SUB)])

        base = wid * chunks_per_w
        pltpu.sync_copy(cols_hbm.at[pl.ds(base, chunks_per_w)], cols_v)
        pltpu.sync_copy(rows_hbm.at[pl.ds(base, chunks_per_w)], rows_v)
        pltpu.sync_copy(vals_hbm.at[pl.ds(base * CH, chunks_per_w * CH)], vals_v)

        plsc.subcore_barrier()

        def chunk_body(c, _):
            pltpu.async_copy(xf_hbm.at[cols_v.at[c]], buf, sem).wait()

            def group_body(g, _):
                vb16 = vals_v[pl.ds(c * CH + g * LANES, LANES)]
                for i in range(LANES):
                    e = g * LANES + i
                    for j in range(F // LANES):
                        sl = pl.ds(j * LANES, LANES)
                        buf[e, sl] = buf[e, sl] * vb16[i]
                return 0

            lax.fori_loop(0, CH // LANES, group_body, 0)

            pltpu.sync_copy(buf, acc.at[rows_v.at[c]], add=True)
            return 0

        lax.fori_loop(0, chunks_per_w, chunk_body, 0)

        plsc.subcore_barrier()

        pltpu.sync_copy(
            acc.at[pl.ds(sid * ROWS_PER_SUB, ROWS_PER_SUB)],
            out_hbm.at[cid, pl.ds(sid * ROWS_PER_SUB, ROWS_PER_SUB)],
        )

    return k(xf, cols2d, rows2d, vals2d, zeros)


def _combine_matmul_body(p_ref, w_ref, o_ref):
    p0 = p_ref[0]
    p1 = p_ref[1]
    o_ref[...] = (
        jnp.dot(p0, w_ref[...], preferred_element_type=jnp.float32,
                precision=lax.Precision.HIGHEST)
        + jnp.dot(p1, w_ref[...], preferred_element_type=jnp.float32,
                  precision=lax.Precision.HIGHEST)
    )


def _combine_matmul(partials, W):
    blk = 1000
    grid = (N // blk,)
    return pl.pallas_call(
        _combine_matmul_body,
        grid=grid,
        in_specs=[
            pl.BlockSpec((NC, blk, F), lambda i: (0, i, 0)),
            pl.BlockSpec((F, F), lambda i: (0, 0)),
        ],
        out_specs=pl.BlockSpec((blk, F), lambda i: (i, 0)),
        out_shape=jax.ShapeDtypeStruct((N, F), jnp.float32),
    )(partials, W)


def kernel(x, a_rows, a_cols, a_vals, W):
    batch, n, f_in = x.shape
    xf = x.reshape(n, f_in)
    nnz = a_rows.shape[0]

    span = NW * CH
    chunks_per_w = -(-nnz // span)
    chunks_per_w = -(-chunks_per_w // 8) * 8
    nnz_pad = span * chunks_per_w
    pad = nnz_pad - nnz
    rows_p = jnp.pad(a_rows, (0, pad))
    cols_p = jnp.pad(a_cols, (0, pad))
    vals_p = jnp.pad(a_vals, (0, pad))

    rows2d = rows_p.reshape(NW * chunks_per_w, CH)
    cols2d = cols_p.reshape(NW * chunks_per_w, CH)
    zeros = jnp.zeros((ROWS_PER_SUB, F), jnp.float32)

    partials = _sc_propagate(xf, cols2d, rows2d, vals_p, zeros, chunks_per_w)
    out = _combine_matmul(partials, W)
    return out.reshape(batch, n, W.shape[1])


# speedup vs baseline: 3.7058x; 1.0001x over previous
import functools

import jax
import jax.numpy as jnp
from jax import lax
from jax.experimental import pallas as pl
from jax.experimental.pallas import tpu as pltpu
from jax.experimental.pallas import tpu_sc as plsc

N = 10000
N_PAD = 10240
F = 128
LANES = 16
NC = 2
NS = 16
NW = NC * NS
CH = 64
NBUF = 4
IB = 40
ROWS_PER_SUB = N_PAD // NS


def _sc_propagate(xf, cols2d, rows2d, vals2d, zeros, chunks_per_w):
    mesh = plsc.VectorSubcoreMesh(core_axis_name="c", subcore_axis_name="s")

    @functools.partial(
        pl.kernel,
        out_type=jax.ShapeDtypeStruct((NC, N_PAD, F), jnp.float32),
        mesh=mesh,
        scratch_types=[
            pltpu.VMEM_SHARED((N_PAD, F), jnp.float32),
            pltpu.VMEM((IB, CH), jnp.int32),
            pltpu.VMEM((IB, CH), jnp.int32),
            pltpu.VMEM((IB * CH,), jnp.float32),
            pltpu.VMEM((NBUF, CH, F), jnp.float32),
            pltpu.SemaphoreType.DMA((NBUF,)),
        ],
    )
    def k(xf_hbm, cols_hbm, rows_hbm, vals_hbm, zeros_hbm, out_hbm,
          acc, cols_v, rows_v, vals_v, buf, sem):
        cid = lax.axis_index("c")
        sid = lax.axis_index("s")
        wid = cid * NS + sid

        pltpu.sync_copy(zeros_hbm, acc.at[pl.ds(sid * ROWS_PER_SUB, ROWS_PER_SUB)])

        base = wid * chunks_per_w

        plsc.subcore_barrier()

        def block_body(nb, _):
            cbase = base + nb * IB
            pltpu.sync_copy(cols_hbm.at[pl.ds(cbase, IB)], cols_v)
            pltpu.sync_copy(rows_hbm.at[pl.ds(cbase, IB)], rows_v)
            pltpu.sync_copy(vals_hbm.at[pl.ds(cbase * CH, IB * CH)], vals_v)

            for b in range(NBUF):
                pltpu.async_copy(xf_hbm.at[cols_v.at[b]], buf.at[b], sem.at[b])

            def chunk_body(c, _):
                p = lax.rem(c, NBUF)
                pltpu.make_async_copy(
                    xf_hbm.at[cols_v.at[c]], buf.at[p], sem.at[p]).wait()

                def group_body(g, _):
                    vb16 = vals_v[pl.ds(c * CH + g * LANES, LANES)]
                    for i in range(LANES):
                        e = g * LANES + i
                        for j in range(F // LANES):
                            sl = pl.ds(j * LANES, LANES)
                            buf[p, e, sl] = buf[p, e, sl] * vb16[i]
                    return 0

                lax.fori_loop(0, CH // LANES, group_body, 0)

                pltpu.sync_copy(buf.at[p], acc.at[rows_v.at[c]], add=True)

                @pl.when(c + NBUF < IB)
                def _():
                    pltpu.async_copy(
                        xf_hbm.at[cols_v.at[c + NBUF]], buf.at[p], sem.at[p])

                return 0

            lax.fori_loop(0, IB, chunk_body, 0)
            return 0

        lax.fori_loop(0, chunks_per_w // IB, block_body, 0)

        plsc.subcore_barrier()

        pltpu.sync_copy(
            acc.at[pl.ds(sid * ROWS_PER_SUB, ROWS_PER_SUB)],
            out_hbm.at[cid, pl.ds(sid * ROWS_PER_SUB, ROWS_PER_SUB)],
        )

    return k(xf, cols2d, rows2d, vals2d, zeros)


def _combine_matmul_body(p_ref, w_ref, o_ref):
    p0 = p_ref[0]
    p1 = p_ref[1]
    o_ref[...] = (
        jnp.dot(p0, w_ref[...], preferred_element_type=jnp.float32,
                precision=lax.Precision.HIGHEST)
        + jnp.dot(p1, w_ref[...], preferred_element_type=jnp.float32,
                  precision=lax.Precision.HIGHEST)
    )


def _combine_matmul(partials, W):
    blk = 1000
    grid = (N // blk,)
    return pl.pallas_call(
        _combine_matmul_body,
        grid=grid,
        in_specs=[
            pl.BlockSpec((NC, blk, F), lambda i: (0, i, 0)),
            pl.BlockSpec((F, F), lambda i: (0, 0)),
        ],
        out_specs=pl.BlockSpec((blk, F), lambda i: (i, 0)),
        out_shape=jax.ShapeDtypeStruct((N, F), jnp.float32),
    )(partials, W)


def kernel(x, a_rows, a_cols, a_vals, W):
    batch, n, f_in = x.shape
    xf = x.reshape(n, f_in)
    nnz = a_rows.shape[0]

    span = NW * CH
    chunks_per_w = -(-nnz // span)
    chunks_per_w = -(-chunks_per_w // 8) * 8
    nnz_pad = span * chunks_per_w
    pad = nnz_pad - nnz
    rows_p = jnp.pad(a_rows, (0, pad))
    cols_p = jnp.pad(a_cols, (0, pad))
    vals_p = jnp.pad(a_vals, (0, pad))

    rows2d = rows_p.reshape(NW * chunks_per_w, CH)
    cols2d = cols_p.reshape(NW * chunks_per_w, CH)
    zeros = jnp.zeros((ROWS_PER_SUB, F), jnp.float32)

    partials = _sc_propagate(xf, cols2d, rows2d, vals_p, zeros, chunks_per_w)
    out = _combine_matmul(partials, W)
    return out.reshape(batch, n, W.shape[1])


# speedup vs baseline: 4.0342x; 1.0886x over previous
import functools

import jax
import jax.numpy as jnp
from jax import lax
from jax.experimental import pallas as pl
from jax.experimental.pallas import tpu as pltpu
from jax.experimental.pallas import tpu_sc as plsc

N = 10000
N_PAD = 10240
F = 128
FH = F // 2
LANES = 16
NC = 2
NS = 16
CH = 128
NBUF = 4
IB = 32
ROWS_PER_SUB = N_PAD // NS


def _sc_propagate(xh, cols2d, rows2d, vals_flat, zeros, chunks_per_t):
    mesh = plsc.VectorSubcoreMesh(core_axis_name="c", subcore_axis_name="s")

    @functools.partial(
        pl.kernel,
        out_type=jax.ShapeDtypeStruct((NC, N_PAD, FH), jnp.float32),
        mesh=mesh,
        scratch_types=[
            pltpu.VMEM_SHARED((N_PAD, FH), jnp.float32),
            pltpu.VMEM_SHARED((N_PAD, FH), jnp.float32),
            pltpu.VMEM((IB, CH), jnp.int32),
            pltpu.VMEM((IB, CH), jnp.int32),
            pltpu.VMEM((IB * CH,), jnp.float32),
            pltpu.VMEM((NBUF, CH, FH), jnp.float32),
            pltpu.SemaphoreType.DMA((NBUF,)),
        ],
        compiler_params=pltpu.CompilerParams(use_tc_tiling_on_sc=False),
    )
    def k(xh_hbm, cols_hbm, rows_hbm, vals_hbm, zeros_hbm, out_hbm,
          xs, acc, cols_v, rows_v, vals_v, buf, sem):
        cid = lax.axis_index("c")
        sid = lax.axis_index("s")

        rsl = pl.ds(sid * ROWS_PER_SUB, ROWS_PER_SUB)
        pltpu.sync_copy(xh_hbm.at[cid, rsl], xs.at[rsl])
        pltpu.sync_copy(zeros_hbm, acc.at[rsl])

        plsc.subcore_barrier()

        base = sid * chunks_per_t

        def block_body(nb, _):
            cbase = base + nb * IB
            pltpu.sync_copy(cols_hbm.at[pl.ds(cbase, IB)], cols_v)
            pltpu.sync_copy(rows_hbm.at[pl.ds(cbase, IB)], rows_v)
            pltpu.sync_copy(vals_hbm.at[pl.ds(cbase * CH, IB * CH)], vals_v)

            for b in range(NBUF):
                pltpu.async_copy(xs.at[cols_v.at[b]], buf.at[b], sem.at[b])

            def chunk_body(c, _):
                p = lax.rem(c, NBUF)
                pltpu.make_async_copy(
                    xs.at[cols_v.at[c]], buf.at[p], sem.at[p]).wait()

                def group_body(g, _):
                    vb16 = vals_v[pl.ds(c * CH + g * LANES, LANES)]
                    for i in range(LANES):
                        e = g * LANES + i
                        for j in range(FH // LANES):
                            sl = pl.ds(j * LANES, LANES)
                            buf[p, e, sl] = buf[p, e, sl] * vb16[i]
                    return 0

                lax.fori_loop(0, CH // LANES, group_body, 0)

                pltpu.sync_copy(buf.at[p], acc.at[rows_v.at[c]], add=True)

                @pl.when(c + NBUF < IB)
                def _():
                    pltpu.async_copy(
                        xs.at[cols_v.at[c + NBUF]], buf.at[p], sem.at[p])

                return 0

            lax.fori_loop(0, IB, chunk_body, 0)
            return 0

        lax.fori_loop(0, chunks_per_t // IB, block_body, 0)

        plsc.subcore_barrier()

        pltpu.sync_copy(acc.at[rsl], out_hbm.at[cid, rsl])

    return k(xh, cols2d, rows2d, vals_flat, zeros)


def _combine_matmul_body(p_ref, w_ref, o_ref):
    o_ref[...] = (
        jnp.dot(p_ref[0], w_ref[pl.ds(0, FH), :],
                preferred_element_type=jnp.float32,
                precision=lax.Precision.HIGHEST)
        + jnp.dot(p_ref[1], w_ref[pl.ds(FH, FH), :],
                  preferred_element_type=jnp.float32,
                  precision=lax.Precision.HIGHEST)
    )


def _combine_matmul(partials, W):
    blk = 1000
    grid = (N // blk,)
    return pl.pallas_call(
        _combine_matmul_body,
        grid=grid,
        in_specs=[
            pl.BlockSpec((NC, blk, FH), lambda i: (0, i, 0)),
            pl.BlockSpec((F, F), lambda i: (0, 0)),
        ],
        out_specs=pl.BlockSpec((blk, F), lambda i: (i, 0)),
        out_shape=jax.ShapeDtypeStruct((N, F), jnp.float32),
    )(partials, W)


def kernel(x, a_rows, a_cols, a_vals, W):
    batch, n, f_in = x.shape
    xf = x.reshape(n, f_in)
    nnz = a_rows.shape[0]

    xh = jnp.pad(xf, ((0, N_PAD - n), (0, 0)))
    xh = xh.reshape(N_PAD, NC, FH).transpose(1, 0, 2)

    span = NS * CH
    chunks_per_t = -(-nnz // span)
    chunks_per_t = -(-chunks_per_t // IB) * IB
    nnz_pad = span * chunks_per_t
    pad = nnz_pad - nnz
    rows_p = jnp.pad(a_rows, (0, pad))
    cols_p = jnp.pad(a_cols, (0, pad))
    vals_p = jnp.pad(a_vals, (0, pad))

    rows2d = rows_p.reshape(NS * chunks_per_t, CH)
    cols2d = cols_p.reshape(NS * chunks_per_t, CH)
    zeros = jnp.zeros((ROWS_PER_SUB, FH), jnp.float32)

    partials = _sc_propagate(xh, cols2d, rows2d, vals_p, zeros, chunks_per_t)
    out = _combine_matmul(partials, W)
    return out.reshape(batch, n, W.shape[1])


# speedup vs baseline: 4.6026x; 1.1409x over previous
import functools

import jax
import jax.numpy as jnp
from jax import lax
from jax.experimental import pallas as pl
from jax.experimental.pallas import tpu as pltpu
from jax.experimental.pallas import tpu_sc as plsc

N = 10000
N_PAD = 10240
F = 128
FH = F // 2
LANES = 16
NC = 2
NS = 16
CH = 128
NBUF = 4
IB = 32
ROWS_PER_SUB = N_PAD // NS


def _sc_propagate(xh, cols2d, rows2d, vals_flat, zeros, chunks_per_t):
    mesh = plsc.VectorSubcoreMesh(core_axis_name="c", subcore_axis_name="s")

    @functools.partial(
        pl.kernel,
        out_type=jax.ShapeDtypeStruct((NC, N_PAD, FH), jnp.float32),
        mesh=mesh,
        scratch_types=[
            pltpu.VMEM_SHARED((N_PAD, FH), jnp.float32),
            pltpu.VMEM_SHARED((N_PAD, FH), jnp.float32),
            pltpu.VMEM((IB, CH), jnp.int32),
            pltpu.VMEM((IB, CH), jnp.int32),
            pltpu.VMEM((IB * CH,), jnp.float32),
            pltpu.VMEM((NBUF, CH, FH), jnp.float32),
            pltpu.SemaphoreType.DMA((NBUF,)),
            pltpu.SemaphoreType.DMA((NBUF,)),
        ],
        compiler_params=pltpu.CompilerParams(use_tc_tiling_on_sc=False),
    )
    def k(xh_hbm, cols_hbm, rows_hbm, vals_hbm, zeros_hbm, out_hbm,
          xs, acc, cols_v, rows_v, vals_v, buf, sem, ssem):
        cid = lax.axis_index("c")
        sid = lax.axis_index("s")

        rsl = pl.ds(sid * ROWS_PER_SUB, ROWS_PER_SUB)
        pltpu.sync_copy(xh_hbm.at[cid, rsl], xs.at[rsl])
        pltpu.sync_copy(zeros_hbm, acc.at[rsl])

        plsc.subcore_barrier()

        base = sid * chunks_per_t

        def block_body(nb, _):
            cbase = base + nb * IB
            pltpu.sync_copy(cols_hbm.at[pl.ds(cbase, IB)], cols_v)
            pltpu.sync_copy(rows_hbm.at[pl.ds(cbase, IB)], rows_v)
            pltpu.sync_copy(vals_hbm.at[pl.ds(cbase * CH, IB * CH)], vals_v)

            for b in range(2):
                pltpu.async_copy(xs.at[cols_v.at[b]], buf.at[b], sem.at[b])

            def chunk_body(c, _):
                p = lax.rem(c, NBUF)
                pltpu.make_async_copy(
                    xs.at[cols_v.at[c]], buf.at[p], sem.at[p]).wait()

                def group_body(g, _):
                    vb16 = vals_v[pl.ds(c * CH + g * LANES, LANES)]
                    for i in range(LANES):
                        e = g * LANES + i
                        for j in range(FH // LANES):
                            sl = pl.ds(j * LANES, LANES)
                            buf[p, e, sl] = buf[p, e, sl] * vb16[i]
                    return 0

                lax.fori_loop(0, CH // LANES, group_body, 0)

                pltpu.async_copy(
                    buf.at[p], acc.at[rows_v.at[c]], ssem.at[p], add=True)

                q = lax.rem(c + 2, NBUF)

                @pl.when((c >= 2) & (c + 2 < IB))
                def _():
                    pltpu.make_async_copy(
                        buf.at[q], acc.at[rows_v.at[c - 2]],
                        ssem.at[q]).wait()
                    pltpu.async_copy(
                        xs.at[cols_v.at[c + 2]], buf.at[q], sem.at[q])

                @pl.when((c < 2) & (c + 2 < IB))
                def _():
                    pltpu.async_copy(
                        xs.at[cols_v.at[c + 2]], buf.at[q], sem.at[q])

                return 0

            lax.fori_loop(0, IB, chunk_body, 0)

            for t in (IB - 4, IB - 3, IB - 2, IB - 1):
                pltpu.make_async_copy(
                    buf.at[t % NBUF], acc.at[rows_v.at[t]],
                    ssem.at[t % NBUF]).wait()
            return 0

        lax.fori_loop(0, chunks_per_t // IB, block_body, 0)

        plsc.subcore_barrier()

        pltpu.sync_copy(acc.at[rsl], out_hbm.at[cid, rsl])

    return k(xh, cols2d, rows2d, vals_flat, zeros)


def _combine_matmul_body(p_ref, w_ref, o_ref):
    o_ref[...] = (
        jnp.dot(p_ref[0], w_ref[pl.ds(0, FH), :],
                preferred_element_type=jnp.float32,
                precision=lax.Precision.HIGHEST)
        + jnp.dot(p_ref[1], w_ref[pl.ds(FH, FH), :],
                  preferred_element_type=jnp.float32,
                  precision=lax.Precision.HIGHEST)
    )


def _combine_matmul(partials, W):
    blk = 1000
    grid = (N // blk,)
    return pl.pallas_call(
        _combine_matmul_body,
        grid=grid,
        in_specs=[
            pl.BlockSpec((NC, blk, FH), lambda i: (0, i, 0)),
            pl.BlockSpec((F, F), lambda i: (0, 0)),
        ],
        out_specs=pl.BlockSpec((blk, F), lambda i: (i, 0)),
        out_shape=jax.ShapeDtypeStruct((N, F), jnp.float32),
    )(partials, W)


def kernel(x, a_rows, a_cols, a_vals, W):
    batch, n, f_in = x.shape
    xf = x.reshape(n, f_in)
    nnz = a_rows.shape[0]

    xh = jnp.pad(xf, ((0, N_PAD - n), (0, 0)))
    xh = xh.reshape(N_PAD, NC, FH).transpose(1, 0, 2)

    span = NS * CH
    chunks_per_t = -(-nnz // span)
    chunks_per_t = -(-chunks_per_t // IB) * IB
    nnz_pad = span * chunks_per_t
    pad = nnz_pad - nnz
    rows_p = jnp.pad(a_rows, (0, pad))
    cols_p = jnp.pad(a_cols, (0, pad))
    vals_p = jnp.pad(a_vals, (0, pad))

    rows2d = rows_p.reshape(NS * chunks_per_t, CH)
    cols2d = cols_p.reshape(NS * chunks_per_t, CH)
    zeros = jnp.zeros((ROWS_PER_SUB, FH), jnp.float32)

    partials = _sc_propagate(xh, cols2d, rows2d, vals_p, zeros, chunks_per_t)
    out = _combine_matmul(partials, W)
    return out.reshape(batch, n, W.shape[1])
